# baseline (speedup 1.0000x reference)
import functools

import jax
import jax.numpy as jnp
from jax import lax
from jax.experimental import pallas as pl
from jax.experimental.pallas import tpu as pltpu
from jax.experimental.pallas import tpu_sc as plsc

N = 100000
E = 1600000
ERM = 13
NRM = 3
RS = 6.0
INV_W = 5.0

NC, NS, L = 2, 16, 16
NW = NC * NS

EC = 1280
EC2 = 2 * EC
N_ECHUNK = E // EC
EGROUPS = EC // L
NODE_CHUNK = 2000
N_NCHUNK = N // NODE_CHUNK
NGROUPS = NODE_CHUNK // L

LN2 = 0.6931471805599453


def _ln16(x):
    ib = plsc.bitcast(x, jnp.int32)
    ex = jnp.right_shift(ib, 23) - 127
    m = plsc.bitcast(
        jnp.bitwise_or(jnp.bitwise_and(ib, 0x007FFFFF), 0x3F800000),
        jnp.float32)
    big = m > 1.4142135381698608
    m = jnp.where(big, m * 0.5, m)
    exf = (ex + jnp.where(big, 1, 0)).astype(jnp.float32)
    s = (m - 1.0) / (m + 1.0)
    s2 = s * s
    lnm = s * (2.0 + s2 * (0.66666666666 + s2 * (0.4 + s2 * 0.2857142857)))
    return exf * LN2 + lnm


def _body(table_hbm, src_hbm, dst_hbm, hop_hbm, bl_hbm, az_hbm, ons_hbm,
          eout_hbm, nout_hbm,
          sraw, draw, sidxe, guard_s, didxe, guard_d, srows, drows,
          eoutv, hopv, hopt, blv, lnr0v, onsv, azv, noutv, sem_s, sem_d):
    wid = lax.axis_index("s") * NC + lax.axis_index("c")
    iota = lax.iota(jnp.int32, L)

    pltpu.sync_copy(hop_hbm, hopv)
    pltpu.sync_copy(bl_hbm, blv)
    pltpu.sync_copy(ons_hbm, onsv)

    for b in range(3):
        a_v = plsc.load_gather(hopv, [b * 26 + 2 * iota])
        c_v = 1.0 + plsc.load_gather(hopv, [b * 26 + 2 * iota + 1])
        plsc.store_scatter(hopt, [b * 32 + iota], a_v)
        plsc.store_scatter(hopt, [b * 32 + 16 + iota], c_v)

    def ginit(j, carry):
        big = jnp.full((L,), 0x7FFFFFF0, jnp.int32)
        guard_s[pl.ds(j * L, L)] = big
        guard_d[pl.ds(j * L, L)] = big
        return carry

    lax.fori_loop(0, EC2 // L, ginit, 0)

    zi_pat = jnp.where(iota >= 2, 1, 0)
    zj_pat = jnp.where(iota >= 1, 1, 0)
    r0v = 0.5 * (plsc.load_gather(blv, [zi_pat]) +
                 plsc.load_gather(blv, [zj_pat]))
    lnr0v[pl.ds(0, L)] = _ln16(r0v)

    def expand(j):
        pos2 = (j * L + iota) * 2
        plsc.store_scatter(sidxe, [pos2], sraw[pl.ds(j * L, L)] * 2)
        plsc.store_scatter(didxe, [pos2], draw[pl.ds(j * L, L)] * 2)

    def egroup(j):
        rowi = j * L + iota
        c0 = jnp.full((L,), 0, jnp.int32)
        sx = plsc.load_gather(srows, [rowi, c0])
        sy = plsc.load_gather(srows, [rowi, c0 + 1])
        sz = plsc.load_gather(srows, [rowi, c0 + 2])
        sw = plsc.load_gather(srows, [rowi, c0 + 3])
        dx = plsc.load_gather(drows, [rowi, c0])
        dy = plsc.load_gather(drows, [rowi, c0 + 1])
        dz = plsc.load_gather(drows, [rowi, c0 + 2])
        dw = plsc.load_gather(drows, [rowi, c0 + 3])
        vx = dx - sx
        vy = dy - sy
        vz = dz - sz
        d2 = vx * vx + vy * vy + vz * vz + 1e-12
        ib = plsc.bitcast(d2, jnp.int32)
        y = plsc.bitcast(0x5F3759DF - jnp.right_shift(ib, 1), jnp.float32)
        hx = 0.5 * d2
        y = y * (1.5 - hx * y * y)
        y = y * (1.5 - hx * y * y)
        y = y * (1.5 - hx * y * y)
        rij = d2 * y
        ln_d2 = _ln16(d2)
        bidx = (sw + dw).astype(jnp.int32)
        lnr0 = plsc.load_gather(lnr0v, [bidx])
        lnratio = lnr0 - 0.5 * ln_d2
        t = jnp.minimum((rij - RS) * INV_W, 85.0)
        fc = 1.0 / (1.0 + jnp.exp(t))
        b32 = jnp.left_shift(bidx, 5)
        for m in range(ERM):
            a = plsc.load_gather(hopt, [b32 + m])
            cc = plsc.load_gather(hopt, [b32 + (16 + m)])
            p = jnp.exp(lnratio * cc)
            plsc.store_scatter(eoutv, [rowi, c0 + m], a * p * fc)

    def echunk(i, carry):
        c = wid + i * NW
        base = c * EC
        pltpu.sync_copy(src_hbm.at[pl.ds(base, EC)], sraw)
        pltpu.sync_copy(dst_hbm.at[pl.ds(base, EC)], draw)
        plsc.parallel_loop(0, EGROUPS, unroll=2)(expand)
        cp_s = pltpu.async_copy(table_hbm.at[sidxe], srows, sem_s)
        cp_d = pltpu.async_copy(table_hbm.at[didxe], drows, sem_d)
        cp_s.wait()
        cp_d.wait()
        plsc.parallel_loop(0, EGROUPS, unroll=2)(egroup)
        pltpu.sync_copy(eoutv, eout_hbm.at[pl.ds(base, EC)])
        return carry

    n_ec = (N_ECHUNK - wid + NW - 1) // NW
    lax.fori_loop(0, n_ec, echunk, 0)

    def ngroup(j):
        rowi = j * L + iota
        az = azv[pl.ds(j * L, L)]
        a3 = az * NRM
        c0 = jnp.full((L,), 0, jnp.int32)
        for m in range(NRM):
            v = plsc.load_gather(onsv, [a3 + m])
            plsc.store_scatter(noutv, [rowi, c0 + m], v)

    def nchunk(i, carry):
        c = wid + i * NW
        base = c * NODE_CHUNK
        pltpu.sync_copy(az_hbm.at[pl.ds(base, NODE_CHUNK)], azv)
        plsc.parallel_loop(0, NGROUPS, unroll=2)(ngroup)
        pltpu.sync_copy(noutv, nout_hbm.at[pl.ds(base, NODE_CHUNK)])
        return carry

    n_nc = (N_NCHUNK - wid + NW - 1) // NW
    lax.fori_loop(0, n_nc, nchunk, 0)


_mesh = plsc.VectorSubcoreMesh(
    core_axis_name="c", subcore_axis_name="s", num_cores=NC, num_subcores=NS)

_sc_call = pl.kernel(
    _body,
    out_type=(
        jax.ShapeDtypeStruct((E, ERM), jnp.float32),
        jax.ShapeDtypeStruct((N, NRM), jnp.float32),
    ),
    mesh=_mesh,
    compiler_params=pltpu.CompilerParams(
        needs_layout_passes=False, use_tc_tiling_on_sc=False),
    scratch_types=[
        pltpu.VMEM((EC,), jnp.int32),
        pltpu.VMEM((EC,), jnp.int32),
        pltpu.VMEM((EC2,), jnp.int32),
        pltpu.VMEM((EC2,), jnp.int32),
        pltpu.VMEM((EC2,), jnp.int32),
        pltpu.VMEM((EC2,), jnp.int32),
        pltpu.VMEM((EC2, 4), jnp.float32),
        pltpu.VMEM((EC2, 4), jnp.float32),
        pltpu.VMEM((EC, ERM), jnp.float32),
        pltpu.VMEM((128,), jnp.float32),
        pltpu.VMEM((128,), jnp.float32),
        pltpu.VMEM((128,), jnp.float32),
        pltpu.VMEM((128,), jnp.float32),
        pltpu.VMEM((128,), jnp.float32),
        pltpu.VMEM((NODE_CHUNK,), jnp.int32),
        pltpu.VMEM((NODE_CHUNK, NRM), jnp.float32),
        pltpu.SemaphoreType.DMA,
        pltpu.SemaphoreType.DMA,
    ],
)


def kernel(positions, hopping_param, onsite_param, bond_length,
           atomic_numbers, edge_index):
    node_table = jnp.concatenate(
        [positions, atomic_numbers.astype(jnp.float32)[:, None]], axis=1)

    def _pad128(x):
        return jnp.concatenate(
            [x, jnp.zeros((128 - x.shape[0],), x.dtype)])

    edge_feats, node_feats = _sc_call(
        node_table,
        edge_index[0],
        edge_index[1],
        _pad128(hopping_param.reshape(-1)),
        _pad128(bond_length),
        atomic_numbers,
        _pad128(onsite_param.reshape(-1)),
    )
    return edge_feats, node_feats


# speedup vs baseline: 1.2933x; 1.2933x over previous
import functools

import jax
import jax.numpy as jnp
from jax import lax
from jax.experimental import pallas as pl
from jax.experimental.pallas import tpu as pltpu
from jax.experimental.pallas import tpu_sc as plsc

N = 100000
E = 1600000
ERM = 13
NRM = 3
RS = 6.0
INV_W = 5.0

NC, NS, L = 2, 16, 16
NW = NC * NS

EC = 1280
EC2 = 2 * EC
N_ECHUNK = E // EC
EGROUPS = EC // L
NODE_CHUNK = 2000
N_NCHUNK = N // NODE_CHUNK
NGROUPS = NODE_CHUNK // L

LN2 = 0.6931471805599453


def _ln16(x):
    ib = plsc.bitcast(x, jnp.int32)
    ex = jnp.right_shift(ib, 23) - 127
    m = plsc.bitcast(
        jnp.bitwise_or(jnp.bitwise_and(ib, 0x007FFFFF), 0x3F800000),
        jnp.float32)
    big = m > 1.4142135381698608
    m = jnp.where(big, m * 0.5, m)
    exf = (ex + jnp.where(big, 1, 0)).astype(jnp.float32)
    s = (m - 1.0) / (m + 1.0)
    s2 = s * s
    lnm = s * (2.0 + s2 * (0.66666666666 + s2 * (0.4 + s2 * 0.2857142857)))
    return exf * LN2 + lnm


def _body(table_hbm, src_hbm, dst_hbm, hop_hbm, bl_hbm, az_hbm, ons_hbm,
          eout_hbm, nout_hbm,
          sraw, draw, sidxe, guard_s, didxe, guard_d, srows, drows,
          eoutv, hopv, hopt, blv, lnr0v, onsv, azv, noutv, sem_s, sem_d):
    wid = lax.axis_index("s") * NC + lax.axis_index("c")
    iota = lax.iota(jnp.int32, L)

    pltpu.sync_copy(hop_hbm, hopv)
    pltpu.sync_copy(bl_hbm, blv)
    pltpu.sync_copy(ons_hbm, onsv)

    for b in range(3):
        a_v = plsc.load_gather(hopv, [b * 26 + 2 * iota])
        c_v = 1.0 + plsc.load_gather(hopv, [b * 26 + 2 * iota + 1])
        plsc.store_scatter(hopt, [b * 32 + iota], a_v)
        plsc.store_scatter(hopt, [b * 32 + 16 + iota], c_v)

    def ginit(j, carry):
        big = jnp.full((L,), 0x7FFFFFF0, jnp.int32)
        guard_s[pl.ds(j * L, L)] = big
        guard_d[pl.ds(j * L, L)] = big
        return carry

    lax.fori_loop(0, EC2 // L, ginit, 0)

    zi_pat = jnp.where(iota >= 2, 1, 0)
    zj_pat = jnp.where(iota >= 1, 1, 0)
    r0v = 0.5 * (plsc.load_gather(blv, [zi_pat]) +
                 plsc.load_gather(blv, [zj_pat]))
    lnr0v[pl.ds(0, L)] = _ln16(r0v)

    def expand(j, carry):
        pos2 = (j * L + iota) * 2
        plsc.store_scatter(sidxe, [pos2], sraw[pl.ds(j * L, L)] * 2)
        plsc.store_scatter(didxe, [pos2], draw[pl.ds(j * L, L)] * 2)
        return carry

    def egroup(j):
        rowi = j * L + iota
        c0 = jnp.full((L,), 0, jnp.int32)
        sx = plsc.load_gather(srows, [rowi, c0])
        sy = plsc.load_gather(srows, [rowi, c0 + 1])
        sz = plsc.load_gather(srows, [rowi, c0 + 2])
        sw = plsc.load_gather(srows, [rowi, c0 + 3])
        dx = plsc.load_gather(drows, [rowi, c0])
        dy = plsc.load_gather(drows, [rowi, c0 + 1])
        dz = plsc.load_gather(drows, [rowi, c0 + 2])
        dw = plsc.load_gather(drows, [rowi, c0 + 3])
        vx = dx - sx
        vy = dy - sy
        vz = dz - sz
        d2 = vx * vx + vy * vy + vz * vz + 1e-12
        ib = plsc.bitcast(d2, jnp.int32)
        y = plsc.bitcast(0x5F3759DF - jnp.right_shift(ib, 1), jnp.float32)
        hx = 0.5 * d2
        y = y * (1.5 - hx * y * y)
        y = y * (1.5 - hx * y * y)
        y = y * (1.5 - hx * y * y)
        rij = d2 * y
        ln_d2 = _ln16(d2)
        bidx = (sw + dw).astype(jnp.int32)
        lnr0 = plsc.load_gather(lnr0v, [bidx])
        lnratio = lnr0 - 0.5 * ln_d2
        t = jnp.minimum((rij - RS) * INV_W, 85.0)
        fc = 1.0 / (1.0 + jnp.exp(t))
        b32 = jnp.left_shift(bidx, 5)
        for m in range(ERM):
            a = plsc.load_gather(hopt, [b32 + m])
            cc = plsc.load_gather(hopt, [b32 + (16 + m)])
            p = jnp.exp(lnratio * cc)
            plsc.store_scatter(eoutv, [rowi, c0 + m], a * p * fc)

    def echunk(i, carry):
        c = wid + i * NW
        base = c * EC
        pltpu.sync_copy(src_hbm.at[pl.ds(base, EC)], sraw)
        pltpu.sync_copy(dst_hbm.at[pl.ds(base, EC)], draw)
        lax.fori_loop(0, EGROUPS, expand, 0)
        cp_s = pltpu.async_copy(table_hbm.at[sidxe], srows, sem_s)
        cp_d = pltpu.async_copy(table_hbm.at[didxe], drows, sem_d)
        cp_s.wait()
        cp_d.wait()

        def epair(jj, carry):
            egroup(jj * 2)
            egroup(jj * 2 + 1)
            return carry

        lax.fori_loop(0, EGROUPS // 2, epair, 0)
        pltpu.sync_copy(eoutv, eout_hbm.at[pl.ds(base, EC)])
        return carry

    n_ec = (N_ECHUNK - wid + NW - 1) // NW
    lax.fori_loop(0, n_ec, echunk, 0)

    def ngroup(j, carry):
        rowi = j * L + iota
        az = azv[pl.ds(j * L, L)]
        a3 = az * NRM
        c0 = jnp.full((L,), 0, jnp.int32)
        for m in range(NRM):
            v = plsc.load_gather(onsv, [a3 + m])
            plsc.store_scatter(noutv, [rowi, c0 + m], v)
        return carry

    def nchunk(i, carry):
        c = wid + i * NW
        base = c * NODE_CHUNK
        pltpu.sync_copy(az_hbm.at[pl.ds(base, NODE_CHUNK)], azv)
        lax.fori_loop(0, NGROUPS, ngroup, 0)
        pltpu.sync_copy(noutv, nout_hbm.at[pl.ds(base, NODE_CHUNK)])
        return carry

    n_nc = (N_NCHUNK - wid + NW - 1) // NW
    lax.fori_loop(0, n_nc, nchunk, 0)


_mesh = plsc.VectorSubcoreMesh(
    core_axis_name="c", subcore_axis_name="s", num_cores=NC, num_subcores=NS)

_sc_call = pl.kernel(
    _body,
    out_type=(
        jax.ShapeDtypeStruct((E, ERM), jnp.float32),
        jax.ShapeDtypeStruct((N, NRM), jnp.float32),
    ),
    mesh=_mesh,
    compiler_params=pltpu.CompilerParams(
        needs_layout_passes=False, use_tc_tiling_on_sc=False),
    scratch_types=[
        pltpu.VMEM((EC,), jnp.int32),
        pltpu.VMEM((EC,), jnp.int32),
        pltpu.VMEM((EC2,), jnp.int32),
        pltpu.VMEM((EC2,), jnp.int32),
        pltpu.VMEM((EC2,), jnp.int32),
        pltpu.VMEM((EC2,), jnp.int32),
        pltpu.VMEM((EC2, 4), jnp.float32),
        pltpu.VMEM((EC2, 4), jnp.float32),
        pltpu.VMEM((EC, ERM), jnp.float32),
        pltpu.VMEM((128,), jnp.float32),
        pltpu.VMEM((128,), jnp.float32),
        pltpu.VMEM((128,), jnp.float32),
        pltpu.VMEM((128,), jnp.float32),
        pltpu.VMEM((128,), jnp.float32),
        pltpu.VMEM((NODE_CHUNK,), jnp.int32),
        pltpu.VMEM((NODE_CHUNK, NRM), jnp.float32),
        pltpu.SemaphoreType.DMA,
        pltpu.SemaphoreType.DMA,
    ],
)


def kernel(positions, hopping_param, onsite_param, bond_length,
           atomic_numbers, edge_index):
    node_table = jnp.concatenate(
        [positions, atomic_numbers.astype(jnp.float32)[:, None]], axis=1)

    def _pad128(x):
        return jnp.concatenate(
            [x, jnp.zeros((128 - x.shape[0],), x.dtype)])

    edge_feats, node_feats = _sc_call(
        node_table,
        edge_index[0],
        edge_index[1],
        _pad128(hopping_param.reshape(-1)),
        _pad128(bond_length),
        atomic_numbers,
        _pad128(onsite_param.reshape(-1)),
    )
    return edge_feats, node_feats


# speedup vs baseline: 1.4526x; 1.1232x over previous
import functools

import jax
import jax.numpy as jnp
from jax import lax
from jax.experimental import pallas as pl
from jax.experimental.pallas import tpu as pltpu
from jax.experimental.pallas import tpu_sc as plsc

N = 100000
E = 1600000
ERM = 13
NRM = 3
RS = 6.0
INV_W = 5.0

NC, NS, L = 2, 16, 16
NW = NC * NS

EC = 1280
EC2 = 2 * EC
N_ECHUNK = E // EC
EGROUPS = EC // L
NODE_CHUNK = 2000
N_NCHUNK = N // NODE_CHUNK
NGROUPS = NODE_CHUNK // L

LN2 = 0.6931471805599453


def _ln16(x):
    ib = plsc.bitcast(x, jnp.int32)
    ex = jnp.right_shift(ib, 23) - 127
    m = plsc.bitcast(
        jnp.bitwise_or(jnp.bitwise_and(ib, 0x007FFFFF), 0x3F800000),
        jnp.float32)
    big = m > 1.4142135381698608
    m = jnp.where(big, m * 0.5, m)
    exf = (ex + jnp.where(big, 1, 0)).astype(jnp.float32)
    s = (m - 1.0) / (m + 1.0)
    s2 = s * s
    lnm = s * (2.0 + s2 * (0.66666666666 + s2 * (0.4 + s2 * 0.2857142857)))
    return exf * LN2 + lnm


def _body(table_hbm, src_hbm, dst_hbm, hop_hbm, bl_hbm, az_hbm, ons_hbm,
          eout_hbm, nout_hbm,
          sraw, draw, sidxe, guard_s, didxe, guard_d, srows, drows,
          eoutv, hopv, hopt, blv, lnr0v, onsv, azv, noutv, sem_s, sem_d):
    wid = lax.axis_index("s") * NC + lax.axis_index("c")
    iota = lax.iota(jnp.int32, L)

    pltpu.sync_copy(hop_hbm, hopv)
    pltpu.sync_copy(bl_hbm, blv)
    pltpu.sync_copy(ons_hbm, onsv)

    for b in range(3):
        a_v = plsc.load_gather(hopv, [b * 26 + 2 * iota])
        c_v = 1.0 + plsc.load_gather(hopv, [b * 26 + 2 * iota + 1])
        plsc.store_scatter(hopt, [b * 32 + iota], a_v)
        plsc.store_scatter(hopt, [b * 32 + 16 + iota], c_v)

    def ginit(j, carry):
        big = jnp.full((L,), 0x7FFFFFF0, jnp.int32)
        guard_s[pl.ds(j * L, L)] = big
        guard_d[pl.ds(j * L, L)] = big
        return carry

    lax.fori_loop(0, EC2 // L, ginit, 0)

    zi_pat = jnp.where(iota >= 2, 1, 0)
    zj_pat = jnp.where(iota >= 1, 1, 0)
    r0v = 0.5 * (plsc.load_gather(blv, [zi_pat]) +
                 plsc.load_gather(blv, [zj_pat]))
    lnr0v[pl.ds(0, L)] = _ln16(r0v)

    def expand(j, carry):
        pos2 = (j * L + iota) * 2
        plsc.store_scatter(sidxe, [pos2], sraw[pl.ds(j * L, L)] * 2)
        plsc.store_scatter(didxe, [pos2], draw[pl.ds(j * L, L)] * 2)
        return carry

    def egroup(j):
        rowi = j * L + iota
        c0 = jnp.full((L,), 0, jnp.int32)
        sx = plsc.load_gather(srows, [rowi, c0])
        sy = plsc.load_gather(srows, [rowi, c0 + 1])
        sz = plsc.load_gather(srows, [rowi, c0 + 2])
        sw = plsc.load_gather(srows, [rowi, c0 + 3])
        dx = plsc.load_gather(drows, [rowi, c0])
        dy = plsc.load_gather(drows, [rowi, c0 + 1])
        dz = plsc.load_gather(drows, [rowi, c0 + 2])
        dw = plsc.load_gather(drows, [rowi, c0 + 3])
        vx = dx - sx
        vy = dy - sy
        vz = dz - sz
        d2 = vx * vx + vy * vy + vz * vz + 1e-12
        ib = plsc.bitcast(d2, jnp.int32)
        y = plsc.bitcast(0x5F3759DF - jnp.right_shift(ib, 1), jnp.float32)
        hx = 0.5 * d2
        y = y * (1.5 - hx * y * y)
        y = y * (1.5 - hx * y * y)
        y = y * (1.5 - hx * y * y)
        rij = d2 * y
        ln_d2 = _ln16(d2)
        bidx = (sw + dw).astype(jnp.int32)
        lnr0 = plsc.load_gather(lnr0v, [bidx])
        lnratio = lnr0 - 0.5 * ln_d2
        t = jnp.minimum((rij - RS) * INV_W, 85.0)
        fc = 1.0 / (1.0 + jnp.exp(t))
        b32 = jnp.left_shift(bidx, 5)
        for m in range(ERM):
            a = plsc.load_gather(hopt, [b32 + m])
            cc = plsc.load_gather(hopt, [b32 + (16 + m)])
            p = jnp.exp(lnratio * cc)
            plsc.store_scatter(eoutv, [rowi, c0 + m], a * p * fc)

    def echunk(i, carry):
        c = wid + i * NW
        base = c * EC
        pltpu.sync_copy(src_hbm.at[pl.ds(base, EC)], sraw)
        pltpu.sync_copy(dst_hbm.at[pl.ds(base, EC)], draw)
        lax.fori_loop(0, EGROUPS, expand, 0)
        cp_s = pltpu.async_copy(table_hbm.at[sidxe], srows, sem_s)
        cp_d = pltpu.async_copy(table_hbm.at[didxe], drows, sem_d)
        cp_s.wait()
        cp_d.wait()

        def eone(jj, carry):
            egroup(jj)
            return carry

        lax.fori_loop(0, EGROUPS, eone, 0)
        pltpu.sync_copy(eoutv, eout_hbm.at[pl.ds(base, EC)])
        return carry

    n_ec = (N_ECHUNK - wid + NW - 1) // NW
    lax.fori_loop(0, n_ec, echunk, 0)

    def ngroup(j, carry):
        rowi = j * L + iota
        az = azv[pl.ds(j * L, L)]
        a3 = az * NRM
        c0 = jnp.full((L,), 0, jnp.int32)
        for m in range(NRM):
            v = plsc.load_gather(onsv, [a3 + m])
            plsc.store_scatter(noutv, [rowi, c0 + m], v)
        return carry

    def nchunk(i, carry):
        c = wid + i * NW
        base = c * NODE_CHUNK
        pltpu.sync_copy(az_hbm.at[pl.ds(base, NODE_CHUNK)], azv)
        lax.fori_loop(0, NGROUPS, ngroup, 0)
        pltpu.sync_copy(noutv, nout_hbm.at[pl.ds(base, NODE_CHUNK)])
        return carry

    n_nc = (N_NCHUNK - wid + NW - 1) // NW
    lax.fori_loop(0, n_nc, nchunk, 0)


_mesh = plsc.VectorSubcoreMesh(
    core_axis_name="c", subcore_axis_name="s", num_cores=NC, num_subcores=NS)

_sc_call = pl.kernel(
    _body,
    out_type=(
        jax.ShapeDtypeStruct((E, ERM), jnp.float32),
        jax.ShapeDtypeStruct((N, NRM), jnp.float32),
    ),
    mesh=_mesh,
    compiler_params=pltpu.CompilerParams(
        needs_layout_passes=False, use_tc_tiling_on_sc=False),
    scratch_types=[
        pltpu.VMEM((EC,), jnp.int32),
        pltpu.VMEM((EC,), jnp.int32),
        pltpu.VMEM((EC2,), jnp.int32),
        pltpu.VMEM((EC2,), jnp.int32),
        pltpu.VMEM((EC2,), jnp.int32),
        pltpu.VMEM((EC2,), jnp.int32),
        pltpu.VMEM((EC2, 4), jnp.float32),
        pltpu.VMEM((EC2, 4), jnp.float32),
        pltpu.VMEM((EC, ERM), jnp.float32),
        pltpu.VMEM((128,), jnp.float32),
        pltpu.VMEM((128,), jnp.float32),
        pltpu.VMEM((128,), jnp.float32),
        pltpu.VMEM((128,), jnp.float32),
        pltpu.VMEM((128,), jnp.float32),
        pltpu.VMEM((NODE_CHUNK,), jnp.int32),
        pltpu.VMEM((NODE_CHUNK, NRM), jnp.float32),
        pltpu.SemaphoreType.DMA,
        pltpu.SemaphoreType.DMA,
    ],
)


def kernel(positions, hopping_param, onsite_param, bond_length,
           atomic_numbers, edge_index):
    node_table = jnp.concatenate(
        [positions, atomic_numbers.astype(jnp.float32)[:, None]], axis=1)

    def _pad128(x):
        return jnp.concatenate(
            [x, jnp.zeros((128 - x.shape[0],), x.dtype)])

    edge_feats, node_feats = _sc_call(
        node_table,
        edge_index[0],
        edge_index[1],
        _pad128(hopping_param.reshape(-1)),
        _pad128(bond_length),
        atomic_numbers,
        _pad128(onsite_param.reshape(-1)),
    )
    return edge_feats, node_feats


# speedup vs baseline: 1.9666x; 1.3539x over previous
import functools

import jax
import jax.numpy as jnp
from jax import lax
from jax.experimental import pallas as pl
from jax.experimental.pallas import tpu as pltpu
from jax.experimental.pallas import tpu_sc as plsc

N = 100000
E = 1600000
ERM = 13
NRM = 3
RS = 6.0
INV_W = 5.0

NC, NS, L = 2, 16, 16
NW = NC * NS

EC = 1280
EC2 = 2 * EC
N_ECHUNK = E // EC
EGROUPS = EC // L
NODE_CHUNK = 2000
N_NCHUNK = N // NODE_CHUNK
NGROUPS = NODE_CHUNK // L

LN2 = 0.6931471805599453


def _ln16(x):
    ib = plsc.bitcast(x, jnp.int32)
    ex = jnp.right_shift(ib, 23) - 127
    m = plsc.bitcast(
        jnp.bitwise_or(jnp.bitwise_and(ib, 0x007FFFFF), 0x3F800000),
        jnp.float32)
    big = m > 1.4142135381698608
    m = jnp.where(big, m * 0.5, m)
    exf = (ex + jnp.where(big, 1, 0)).astype(jnp.float32)
    s = (m - 1.0) / (m + 1.0)
    s2 = s * s
    lnm = s * (2.0 + s2 * (0.66666666666 + s2 * (0.4 + s2 * 0.2857142857)))
    return exf * LN2 + lnm


def _body(table_hbm, src_hbm, dst_hbm, hop_hbm, bl_hbm, az_hbm, ons_hbm,
          eout_hbm, nout_hbm,
          sraw, draw, sidxe, guard_s, didxe, guard_d, srows, drows,
          eoutv, hopv, blv, lnr0v, onsv, azv, noutv, sem_s, sem_d):
    wid = lax.axis_index("s") * NC + lax.axis_index("c")
    iota = lax.iota(jnp.int32, L)

    pltpu.sync_copy(hop_hbm, hopv)
    pltpu.sync_copy(bl_hbm, blv)
    pltpu.sync_copy(ons_hbm, onsv)


    def ginit(j, carry):
        big = jnp.full((L,), 0x7FFFFFF0, jnp.int32)
        guard_s[pl.ds(j * L, L)] = big
        guard_d[pl.ds(j * L, L)] = big
        return carry

    lax.fori_loop(0, EC2 // L, ginit, 0)

    zi_pat = jnp.where(iota >= 2, 1, 0)
    zj_pat = jnp.where(iota >= 1, 1, 0)
    r0v = 0.5 * (plsc.load_gather(blv, [zi_pat]) +
                 plsc.load_gather(blv, [zj_pat]))
    lnr0v[pl.ds(0, L)] = _ln16(r0v)

    def expand(j, carry):
        pos2 = (j * L + iota) * 2
        plsc.store_scatter(sidxe, [pos2], sraw[pl.ds(j * L, L)] * 2)
        plsc.store_scatter(didxe, [pos2], draw[pl.ds(j * L, L)] * 2)
        return carry

    def egroup(j):
        rowi = j * L + iota
        c0 = jnp.full((L,), 0, jnp.int32)
        sx = plsc.load_gather(srows, [rowi, c0])
        sy = plsc.load_gather(srows, [rowi, c0 + 1])
        sz = plsc.load_gather(srows, [rowi, c0 + 2])
        sw = plsc.load_gather(srows, [rowi, c0 + 3])
        dx = plsc.load_gather(drows, [rowi, c0])
        dy = plsc.load_gather(drows, [rowi, c0 + 1])
        dz = plsc.load_gather(drows, [rowi, c0 + 2])
        dw = plsc.load_gather(drows, [rowi, c0 + 3])
        vx = dx - sx
        vy = dy - sy
        vz = dz - sz
        d2 = vx * vx + vy * vy + vz * vz + 1e-12
        ib = plsc.bitcast(d2, jnp.int32)
        y = plsc.bitcast(0x5F3759DF - jnp.right_shift(ib, 1), jnp.float32)
        hx = 0.5 * d2
        y = y * (1.5 - hx * y * y)
        y = y * (1.5 - hx * y * y)
        y = y * (1.5 - hx * y * y)
        rij = d2 * y
        ln_d2 = _ln16(d2)
        bidx = (sw + dw).astype(jnp.int32)
        lnr0 = plsc.load_gather(lnr0v, [bidx])
        lnratio = lnr0 - 0.5 * ln_d2
        t = jnp.minimum((rij - RS) * INV_W, 85.0)
        fc = 1.0 / (1.0 + jnp.exp(t))
        b26 = bidx * 26
        for m in range(ERM):
            a = plsc.load_gather(hopv, [b26 + (2 * m)])
            b = plsc.load_gather(hopv, [b26 + (2 * m + 1)])
            p = jnp.exp(lnratio * (1.0 + b))
            plsc.store_scatter(eoutv, [rowi, c0 + m], a * p * fc)

    def echunk(i, carry):
        c = wid + i * NW
        base = c * EC
        pltpu.sync_copy(src_hbm.at[pl.ds(base, EC)], sraw)
        pltpu.sync_copy(dst_hbm.at[pl.ds(base, EC)], draw)
        lax.fori_loop(0, EGROUPS, expand, 0)
        cp_s = pltpu.async_copy(table_hbm.at[sidxe], srows, sem_s)
        cp_d = pltpu.async_copy(table_hbm.at[didxe], drows, sem_d)
        cp_s.wait()
        cp_d.wait()

        def eone(jj, carry):
            egroup(jj)
            return carry

        lax.fori_loop(0, EGROUPS, eone, 0)
        pltpu.sync_copy(eoutv, eout_hbm.at[pl.ds(base, EC)])
        return carry

    n_ec = (N_ECHUNK - wid + NW - 1) // NW
    lax.fori_loop(0, n_ec, echunk, 0)

    def ngroup(j, carry):
        rowi = j * L + iota
        az = azv[pl.ds(j * L, L)]
        a3 = az * NRM
        c0 = jnp.full((L,), 0, jnp.int32)
        for m in range(NRM):
            v = plsc.load_gather(onsv, [a3 + m])
            plsc.store_scatter(noutv, [rowi, c0 + m], v)
        return carry

    def nchunk(i, carry):
        c = wid + i * NW
        base = c * NODE_CHUNK
        pltpu.sync_copy(az_hbm.at[pl.ds(base, NODE_CHUNK)], azv)
        lax.fori_loop(0, NGROUPS, ngroup, 0)
        pltpu.sync_copy(noutv, nout_hbm.at[pl.ds(base, NODE_CHUNK)])
        return carry

    n_nc = (N_NCHUNK - wid + NW - 1) // NW
    lax.fori_loop(0, n_nc, nchunk, 0)


_mesh = plsc.VectorSubcoreMesh(
    core_axis_name="c", subcore_axis_name="s", num_cores=NC, num_subcores=NS)

_sc_call = pl.kernel(
    _body,
    out_type=(
        jax.ShapeDtypeStruct((E, ERM), jnp.float32),
        jax.ShapeDtypeStruct((N, NRM), jnp.float32),
    ),
    mesh=_mesh,
    compiler_params=pltpu.CompilerParams(
        needs_layout_passes=False, use_tc_tiling_on_sc=False),
    scratch_types=[
        pltpu.VMEM((EC,), jnp.int32),
        pltpu.VMEM((EC,), jnp.int32),
        pltpu.VMEM((EC2,), jnp.int32),
        pltpu.VMEM((EC2,), jnp.int32),
        pltpu.VMEM((EC2,), jnp.int32),
        pltpu.VMEM((EC2,), jnp.int32),
        pltpu.VMEM((EC2, 4), jnp.float32),
        pltpu.VMEM((EC2, 4), jnp.float32),
        pltpu.VMEM((EC, ERM), jnp.float32),
        pltpu.VMEM((128,), jnp.float32),
        pltpu.VMEM((128,), jnp.float32),
        pltpu.VMEM((128,), jnp.float32),
        pltpu.VMEM((128,), jnp.float32),
        pltpu.VMEM((NODE_CHUNK,), jnp.int32),
        pltpu.VMEM((NODE_CHUNK, NRM), jnp.float32),
        pltpu.SemaphoreType.DMA,
        pltpu.SemaphoreType.DMA,
    ],
)


def kernel(positions, hopping_param, onsite_param, bond_length,
           atomic_numbers, edge_index):
    node_table = jnp.concatenate(
        [positions, atomic_numbers.astype(jnp.float32)[:, None]], axis=1)

    def _pad128(x):
        return jnp.concatenate(
            [x, jnp.zeros((128 - x.shape[0],), x.dtype)])

    edge_feats, node_feats = _sc_call(
        node_table,
        edge_index[0],
        edge_index[1],
        _pad128(hopping_param.reshape(-1)),
        _pad128(bond_length),
        atomic_numbers,
        _pad128(onsite_param.reshape(-1)),
    )
    return edge_feats, node_feats
